# baseline (device time: 39616 ns/iter reference)
import jax
import jax.numpy as jnp
from jax import lax
from jax.experimental import pallas as pl
from jax.experimental.pallas import tpu as pltpu

B, SQ, H, D = 8, 8, 16, 128
SKV = 1024
SCALE = D ** -0.5


def _local_body(r_ref, q_hbm, k_hbm, v_hbm, o_ref, m_ref, l_ref,
                qbuf, kbuf, vbuf, qsem, ksem, vsem):
    r = r_ref[0]
    h = pl.program_id(0)
    slot = h % 2

    def make_dmas(hh, sl):
        q_dma = pltpu.make_async_copy(
            q_hbm.at[r, :, hh, :], qbuf.at[sl], qsem.at[sl])
        k_dma = pltpu.make_async_copy(
            k_hbm.at[r, :, hh, :], kbuf.at[sl], ksem.at[sl])
        v_dma = pltpu.make_async_copy(
            v_hbm.at[r, :, hh, :], vbuf.at[sl], vsem.at[sl])
        return q_dma, k_dma, v_dma

    @pl.when(h == 0)
    def _():
        for dma in make_dmas(h, slot):
            dma.start()

    @pl.when(h + 1 < H)
    def _():
        for dma in make_dmas(h + 1, 1 - slot):
            dma.start()

    for dma in make_dmas(h, slot):
        dma.wait()

    q = qbuf[slot].astype(jnp.bfloat16)
    k = kbuf[slot].astype(jnp.bfloat16)
    s = lax.dot_general(
        q, k, (((1,), (1,)), ((), ())),
        preferred_element_type=jnp.float32) * SCALE
    m = jnp.max(s, axis=1, keepdims=True)
    p = jnp.exp(s - m)
    l = jnp.sum(p, axis=1, keepdims=True)
    v = vbuf[slot].astype(jnp.bfloat16)
    o = lax.dot_general(
        p.astype(jnp.bfloat16), v, (((1,), (0,)), ((), ())),
        preferred_element_type=jnp.float32)
    o_ref[pl.ds(h, 1)] = o[None]
    m_ref[pl.ds(h, 1)] = m[None]
    l_ref[pl.ds(h, 1)] = l[None]


def _local_partial(Q, K, V, rid):
    grid_spec = pltpu.PrefetchScalarGridSpec(
        num_scalar_prefetch=1,
        grid=(H,),
        in_specs=[
            pl.BlockSpec(memory_space=pltpu.MemorySpace.HBM),
            pl.BlockSpec(memory_space=pltpu.MemorySpace.HBM),
            pl.BlockSpec(memory_space=pltpu.MemorySpace.HBM),
        ],
        out_specs=[
            pl.BlockSpec((H, SQ, D), lambda i, r: (0, 0, 0)),
            pl.BlockSpec((H, SQ, 1), lambda i, r: (0, 0, 0)),
            pl.BlockSpec((H, SQ, 1), lambda i, r: (0, 0, 0)),
        ],
        scratch_shapes=[
            pltpu.VMEM((2, SQ, D), jnp.float32),
            pltpu.VMEM((2, SKV, D), jnp.float32),
            pltpu.VMEM((2, SKV, D), jnp.float32),
            pltpu.SemaphoreType.DMA((2,)),
            pltpu.SemaphoreType.DMA((2,)),
            pltpu.SemaphoreType.DMA((2,)),
        ],
    )
    return pl.pallas_call(
        _local_body,
        grid_spec=grid_spec,
        out_shape=[
            jax.ShapeDtypeStruct((H, SQ, D), jnp.float32),
            jax.ShapeDtypeStruct((H, SQ, 1), jnp.float32),
            jax.ShapeDtypeStruct((H, SQ, 1), jnp.float32),
        ],
    )(rid, Q, K, V)


def _combine_body(o_ref, m_ref, l_ref, out_ref,
                  zsend_o, ml_acc, zrecv_o, zrecv_ml,
                  z_send_o_sem, z_recv_o_sem, z_send_ml_sem, z_recv_ml_sem,
                  xy_send_sem, xy_recv_sem):
    mx = lax.axis_index("x")
    my = lax.axis_index("y")
    mz = lax.axis_index("z")
    rid = mx * 4 + my

    z_partners = [(mx, my, mz ^ d) for d in range(1, 4)]
    xy_partners = [(mx, my ^ d, mz) for d in range(1, 4)] + [(mx ^ 1, my, mz)]
    barrier = pltpu.get_barrier_semaphore()
    for dev in z_partners + xy_partners:
        pl.semaphore_signal(barrier, inc=1, device_id=dev,
                            device_id_type=pl.DeviceIdType.MESH)
    pl.semaphore_wait(barrier, len(z_partners) + len(xy_partners))

    zsend_o[...] = o_ref[...].astype(jnp.bfloat16)
    ml_acc[0] = m_ref[...]
    ml_acc[1] = l_ref[...]
    z_rdmas = []
    for d in range(1, 4):
        dev = (mx, my, mz ^ d)
        rdma_o = pltpu.make_async_remote_copy(
            src_ref=zsend_o, dst_ref=zrecv_o.at[d - 1],
            send_sem=z_send_o_sem.at[d - 1], recv_sem=z_recv_o_sem.at[d - 1],
            device_id=dev, device_id_type=pl.DeviceIdType.MESH)
        rdma_ml = pltpu.make_async_remote_copy(
            src_ref=ml_acc, dst_ref=zrecv_ml.at[d - 1],
            send_sem=z_send_ml_sem.at[d - 1], recv_sem=z_recv_ml_sem.at[d - 1],
            device_id=dev, device_id_type=pl.DeviceIdType.MESH)
        rdma_o.start()
        rdma_ml.start()
        z_rdmas.append((rdma_o, rdma_ml))
    for rdma_o, rdma_ml in z_rdmas:
        rdma_ml.wait()
        rdma_o.wait()

    m_tot = jnp.maximum(ml_acc[0], jnp.maximum(
        jnp.maximum(zrecv_ml[0, 0], zrecv_ml[1, 0]), zrecv_ml[2, 0]))
    a_self = jnp.exp(ml_acc[0] - m_tot)
    acc = o_ref[...] * a_self
    l_tot = ml_acc[1] * a_self
    for d in range(3):
        a_d = jnp.exp(zrecv_ml[d, 0] - m_tot)
        acc = acc + zrecv_o[d].astype(jnp.float32) * a_d
        l_tot = l_tot + zrecv_ml[d, 1] * a_d

    final = acc / l_tot
    out_ref[pl.ds(rid, 1)] = jnp.swapaxes(final, 0, 1)[None]

    y_rdmas = []
    for d in range(1, 4):
        rdma = pltpu.make_async_remote_copy(
            src_ref=out_ref.at[pl.ds(rid, 1)],
            dst_ref=out_ref.at[pl.ds(rid, 1)],
            send_sem=xy_send_sem.at[d - 1], recv_sem=xy_recv_sem.at[d - 1],
            device_id=(mx, my ^ d, mz), device_id_type=pl.DeviceIdType.MESH)
        rdma.start()
        y_rdmas.append(rdma)
    for rdma in y_rdmas:
        rdma.wait()

    row = mx * 4
    rdma = pltpu.make_async_remote_copy(
        src_ref=out_ref.at[pl.ds(row, 4)],
        dst_ref=out_ref.at[pl.ds(row, 4)],
        send_sem=xy_send_sem.at[3], recv_sem=xy_recv_sem.at[3],
        device_id=(mx ^ 1, my, mz), device_id_type=pl.DeviceIdType.MESH)
    rdma.start()
    rdma.wait()


def _combine(o_part, m_part, l_part):
    return pl.pallas_call(
        _combine_body,
        in_specs=[
            pl.BlockSpec(memory_space=pltpu.VMEM),
            pl.BlockSpec(memory_space=pltpu.VMEM),
            pl.BlockSpec(memory_space=pltpu.VMEM),
        ],
        out_specs=pl.BlockSpec(memory_space=pltpu.VMEM),
        out_shape=jax.ShapeDtypeStruct((B, SQ, H, D), jnp.float32),
        scratch_shapes=[
            pltpu.VMEM((H, SQ, D), jnp.bfloat16),
            pltpu.VMEM((2, H, SQ, 1), jnp.float32),
            pltpu.VMEM((3, H, SQ, D), jnp.bfloat16),
            pltpu.VMEM((3, 2, H, SQ, 1), jnp.float32),
            pltpu.SemaphoreType.DMA((3,)),
            pltpu.SemaphoreType.DMA((3,)),
            pltpu.SemaphoreType.DMA((3,)),
            pltpu.SemaphoreType.DMA((3,)),
            pltpu.SemaphoreType.DMA((4,)),
            pltpu.SemaphoreType.DMA((4,)),
        ],
        compiler_params=pltpu.CompilerParams(collective_id=0),
    )(o_part, m_part, l_part)


def kernel(Q, K, V):
    rid = (lax.axis_index("x") * 4 + lax.axis_index("y")).astype(jnp.int32)
    rid_arr = jnp.reshape(rid, (1,))
    o_part, m_part, l_part = _local_partial(Q, K, V, rid_arr)
    return _combine(o_part, m_part, l_part)


# device time: 39502 ns/iter; 1.0029x vs baseline; 1.0029x over previous
import jax
import jax.numpy as jnp
from jax import lax
from jax.experimental import pallas as pl
from jax.experimental.pallas import tpu as pltpu

B, SQ, H, D = 8, 8, 16, 128
SKV = 1024
SCALE = D ** -0.5


def _body(r_ref, q_hbm, k_hbm, v_hbm, out_ref,
          qbuf, kbuf, vbuf, o_acc, m_acc, l_acc,
          zsend_o, ml_acc, zrecv_o, zrecv_ml,
          qsem, ksem, vsem,
          z_send_o_sem, z_recv_o_sem, z_send_ml_sem, z_recv_ml_sem,
          xy_send_sem, xy_recv_sem):
    r = r_ref[0]
    h = pl.program_id(0)
    slot = h % 2

    def make_dmas(hh, sl):
        q_dma = pltpu.make_async_copy(
            q_hbm.at[r, :, hh, :], qbuf.at[sl], qsem.at[sl])
        k_dma = pltpu.make_async_copy(
            k_hbm.at[r, :, hh, :], kbuf.at[sl], ksem.at[sl])
        v_dma = pltpu.make_async_copy(
            v_hbm.at[r, :, hh, :], vbuf.at[sl], vsem.at[sl])
        return q_dma, k_dma, v_dma

    @pl.when(h == 0)
    def _():
        for dma in make_dmas(h, slot):
            dma.start()

    @pl.when(h + 1 < H)
    def _():
        for dma in make_dmas(h + 1, 1 - slot):
            dma.start()

    for dma in make_dmas(h, slot):
        dma.wait()

    q = qbuf[slot].astype(jnp.bfloat16)
    k = kbuf[slot].astype(jnp.bfloat16)
    s = lax.dot_general(
        q, k, (((1,), (1,)), ((), ())),
        preferred_element_type=jnp.float32) * SCALE
    m = jnp.max(s, axis=1, keepdims=True)
    p = jnp.exp(s - m)
    l = jnp.sum(p, axis=1, keepdims=True)
    v = vbuf[slot].astype(jnp.bfloat16)
    o = lax.dot_general(
        p.astype(jnp.bfloat16), v, (((1,), (0,)), ((), ())),
        preferred_element_type=jnp.float32)
    o_acc[pl.ds(h, 1)] = o[None]
    m_acc[pl.ds(h, 1)] = m[None]
    l_acc[pl.ds(h, 1)] = l[None]

    @pl.when(h == H - 1)
    def _combine():
        mx = lax.axis_index("x")
        my = lax.axis_index("y")
        mz = lax.axis_index("z")
        rid = mx * 4 + my

        partners = ([(mx, my, mz ^ d) for d in range(1, 4)]
                    + [(mx, my ^ d, mz) for d in range(1, 4)]
                    + [(mx ^ 1, my, mz)])
        barrier = pltpu.get_barrier_semaphore()
        for dev in partners:
            pl.semaphore_signal(barrier, inc=1, device_id=dev,
                                device_id_type=pl.DeviceIdType.MESH)
        pl.semaphore_wait(barrier, len(partners))

        zsend_o[...] = o_acc[...].astype(jnp.bfloat16)
        ml_acc[0] = m_acc[...]
        ml_acc[1] = l_acc[...]
        z_rdmas = []
        for d in range(1, 4):
            dev = (mx, my, mz ^ d)
            rdma_o = pltpu.make_async_remote_copy(
                src_ref=zsend_o, dst_ref=zrecv_o.at[d - 1],
                send_sem=z_send_o_sem.at[d - 1],
                recv_sem=z_recv_o_sem.at[d - 1],
                device_id=dev, device_id_type=pl.DeviceIdType.MESH)
            rdma_ml = pltpu.make_async_remote_copy(
                src_ref=ml_acc, dst_ref=zrecv_ml.at[d - 1],
                send_sem=z_send_ml_sem.at[d - 1],
                recv_sem=z_recv_ml_sem.at[d - 1],
                device_id=dev, device_id_type=pl.DeviceIdType.MESH)
            rdma_o.start()
            rdma_ml.start()
            z_rdmas.append((rdma_o, rdma_ml))
        for rdma_o, rdma_ml in z_rdmas:
            rdma_ml.wait()
            rdma_o.wait()

        m_tot = jnp.maximum(ml_acc[0], jnp.maximum(
            jnp.maximum(zrecv_ml[0, 0], zrecv_ml[1, 0]), zrecv_ml[2, 0]))
        a_self = jnp.exp(ml_acc[0] - m_tot)
        acc = o_acc[...] * a_self
        l_tot = ml_acc[1] * a_self
        for d in range(3):
            a_d = jnp.exp(zrecv_ml[d, 0] - m_tot)
            acc = acc + zrecv_o[d].astype(jnp.float32) * a_d
            l_tot = l_tot + zrecv_ml[d, 1] * a_d

        final = acc / l_tot
        out_ref[pl.ds(rid, 1)] = jnp.swapaxes(final, 0, 1)[None]

        y_rdmas = []
        for d in range(1, 4):
            rdma = pltpu.make_async_remote_copy(
                src_ref=out_ref.at[pl.ds(rid, 1)],
                dst_ref=out_ref.at[pl.ds(rid, 1)],
                send_sem=xy_send_sem.at[d - 1], recv_sem=xy_recv_sem.at[d - 1],
                device_id=(mx, my ^ d, mz),
                device_id_type=pl.DeviceIdType.MESH)
            rdma.start()
            y_rdmas.append(rdma)
        for rdma in y_rdmas:
            rdma.wait()

        row = mx * 4
        rdma = pltpu.make_async_remote_copy(
            src_ref=out_ref.at[pl.ds(row, 4)],
            dst_ref=out_ref.at[pl.ds(row, 4)],
            send_sem=xy_send_sem.at[3], recv_sem=xy_recv_sem.at[3],
            device_id=(mx ^ 1, my, mz), device_id_type=pl.DeviceIdType.MESH)
        rdma.start()
        rdma.wait()


def kernel(Q, K, V):
    rid = (lax.axis_index("x") * 4 + lax.axis_index("y")).astype(jnp.int32)
    rid_arr = jnp.reshape(rid, (1,))
    grid_spec = pltpu.PrefetchScalarGridSpec(
        num_scalar_prefetch=1,
        grid=(H,),
        in_specs=[
            pl.BlockSpec(memory_space=pltpu.MemorySpace.HBM),
            pl.BlockSpec(memory_space=pltpu.MemorySpace.HBM),
            pl.BlockSpec(memory_space=pltpu.MemorySpace.HBM),
        ],
        out_specs=pl.BlockSpec((B, SQ, H, D), lambda i, r: (0, 0, 0, 0)),
        scratch_shapes=[
            pltpu.VMEM((2, SQ, D), jnp.float32),
            pltpu.VMEM((2, SKV, D), jnp.float32),
            pltpu.VMEM((2, SKV, D), jnp.float32),
            pltpu.VMEM((H, SQ, D), jnp.float32),
            pltpu.VMEM((H, SQ, 1), jnp.float32),
            pltpu.VMEM((H, SQ, 1), jnp.float32),
            pltpu.VMEM((H, SQ, D), jnp.bfloat16),
            pltpu.VMEM((2, H, SQ, 1), jnp.float32),
            pltpu.VMEM((3, H, SQ, D), jnp.bfloat16),
            pltpu.VMEM((3, 2, H, SQ, 1), jnp.float32),
            pltpu.SemaphoreType.DMA((2,)),
            pltpu.SemaphoreType.DMA((2,)),
            pltpu.SemaphoreType.DMA((2,)),
            pltpu.SemaphoreType.DMA((3,)),
            pltpu.SemaphoreType.DMA((3,)),
            pltpu.SemaphoreType.DMA((3,)),
            pltpu.SemaphoreType.DMA((3,)),
            pltpu.SemaphoreType.DMA((4,)),
            pltpu.SemaphoreType.DMA((4,)),
        ],
    )
    return pl.pallas_call(
        _body,
        grid_spec=grid_spec,
        out_shape=jax.ShapeDtypeStruct((B, SQ, H, D), jnp.float32),
        compiler_params=pltpu.CompilerParams(collective_id=0),
    )(rid_arr, Q, K, V)


# device time: 38362 ns/iter; 1.0327x vs baseline; 1.0297x over previous
import jax
import jax.numpy as jnp
from jax import lax
from jax.experimental import pallas as pl
from jax.experimental.pallas import tpu as pltpu

B, SQ, H, D = 8, 8, 16, 128
SKV = 1024
SCALE = D ** -0.5


def _body(r_ref, q_hbm, k_hbm, v_hbm, out_ref,
          qbuf, kbuf, vbuf, o_acc, m_acc, l_acc,
          zsend_o, ml_acc, zrecv_o, zrecv_ml,
          qsem, ksem, vsem,
          z_send_o_sem, z_recv_o_sem, z_send_ml_sem, z_recv_ml_sem,
          xy_send_sem, xy_recv_sem):
    r = r_ref[0]
    h = pl.program_id(0)
    slot = h % 2

    HALF = SKV // 2

    def make_dmas(hh, sl):
        dmas = [pltpu.make_async_copy(
            q_hbm.at[r, :, hh, :], qbuf.at[sl], qsem.at[sl])]
        for half in range(2):
            rows = pl.ds(half * HALF, HALF)
            dmas.append(pltpu.make_async_copy(
                k_hbm.at[r, rows, hh, :], kbuf.at[sl, rows],
                ksem.at[sl, half]))
            dmas.append(pltpu.make_async_copy(
                v_hbm.at[r, rows, hh, :], vbuf.at[sl, rows],
                vsem.at[sl, half]))
        return dmas

    @pl.when(h == 0)
    def _():
        for dma in make_dmas(h, slot):
            dma.start()

    @pl.when(h + 1 < H)
    def _():
        for dma in make_dmas(h + 1, 1 - slot):
            dma.start()

    for dma in make_dmas(h, slot):
        dma.wait()

    q = qbuf[slot].astype(jnp.bfloat16)
    k = kbuf[slot].astype(jnp.bfloat16)
    s = lax.dot_general(
        q, k, (((1,), (1,)), ((), ())),
        preferred_element_type=jnp.float32) * SCALE
    m = jnp.max(s, axis=1, keepdims=True)
    p = jnp.exp(s - m)
    l = jnp.sum(p, axis=1, keepdims=True)
    v = vbuf[slot].astype(jnp.bfloat16)
    o = lax.dot_general(
        p.astype(jnp.bfloat16), v, (((1,), (0,)), ((), ())),
        preferred_element_type=jnp.float32)
    o_acc[pl.ds(h, 1)] = o[None]
    m_acc[pl.ds(h, 1)] = m[None]
    l_acc[pl.ds(h, 1)] = l[None]

    @pl.when(h == H - 1)
    def _combine():
        mx = lax.axis_index("x")
        my = lax.axis_index("y")
        mz = lax.axis_index("z")
        rid = mx * 4 + my

        partners = ([(mx, my, mz ^ d) for d in range(1, 4)]
                    + [(mx ^ (t >> 2), my ^ (t & 3), mz) for t in range(1, 8)])
        barrier = pltpu.get_barrier_semaphore()
        for dev in partners:
            pl.semaphore_signal(barrier, inc=1, device_id=dev,
                                device_id_type=pl.DeviceIdType.MESH)
        pl.semaphore_wait(barrier, len(partners))

        zsend_o[...] = o_acc[...].astype(jnp.bfloat16)
        ml_acc[0] = m_acc[...]
        ml_acc[1] = l_acc[...]
        z_rdmas = []
        for d in range(1, 4):
            dev = (mx, my, mz ^ d)
            rdma_o = pltpu.make_async_remote_copy(
                src_ref=zsend_o, dst_ref=zrecv_o.at[d - 1],
                send_sem=z_send_o_sem.at[d - 1],
                recv_sem=z_recv_o_sem.at[d - 1],
                device_id=dev, device_id_type=pl.DeviceIdType.MESH)
            rdma_ml = pltpu.make_async_remote_copy(
                src_ref=ml_acc, dst_ref=zrecv_ml.at[d - 1],
                send_sem=z_send_ml_sem.at[d - 1],
                recv_sem=z_recv_ml_sem.at[d - 1],
                device_id=dev, device_id_type=pl.DeviceIdType.MESH)
            rdma_o.start()
            rdma_ml.start()
            z_rdmas.append((rdma_o, rdma_ml))
        for rdma_o, rdma_ml in z_rdmas:
            rdma_ml.wait()
            rdma_o.wait()

        m_tot = jnp.maximum(ml_acc[0], jnp.maximum(
            jnp.maximum(zrecv_ml[0, 0], zrecv_ml[1, 0]), zrecv_ml[2, 0]))
        a_self = jnp.exp(ml_acc[0] - m_tot)
        acc = o_acc[...] * a_self
        l_tot = ml_acc[1] * a_self
        for d in range(3):
            a_d = jnp.exp(zrecv_ml[d, 0] - m_tot)
            acc = acc + zrecv_o[d].astype(jnp.float32) * a_d
            l_tot = l_tot + zrecv_ml[d, 1] * a_d

        final = acc / l_tot
        out_ref[pl.ds(rid, 1)] = jnp.swapaxes(final, 0, 1)[None]

        _xy_phase(out_ref, rid, mx, my, mz, xy_send_sem, xy_recv_sem)


def _xy_phase(out_ref, rid, mx, my, mz, xy_send_sem, xy_recv_sem):
    xy_rdmas = []
    for t in range(1, 8):
        rdma = pltpu.make_async_remote_copy(
            src_ref=out_ref.at[pl.ds(rid, 1)],
            dst_ref=out_ref.at[pl.ds(rid, 1)],
            send_sem=xy_send_sem.at[t - 1], recv_sem=xy_recv_sem.at[t - 1],
            device_id=(mx ^ (t >> 2), my ^ (t & 3), mz),
            device_id_type=pl.DeviceIdType.MESH)
        rdma.start()
        xy_rdmas.append(rdma)
    for rdma in xy_rdmas:
        rdma.wait()


def kernel(Q, K, V):
    rid = (lax.axis_index("x") * 4 + lax.axis_index("y")).astype(jnp.int32)
    rid_arr = jnp.reshape(rid, (1,))
    grid_spec = pltpu.PrefetchScalarGridSpec(
        num_scalar_prefetch=1,
        grid=(H,),
        in_specs=[
            pl.BlockSpec(memory_space=pltpu.MemorySpace.HBM),
            pl.BlockSpec(memory_space=pltpu.MemorySpace.HBM),
            pl.BlockSpec(memory_space=pltpu.MemorySpace.HBM),
        ],
        out_specs=pl.BlockSpec((B, SQ, H, D), lambda i, r: (0, 0, 0, 0)),
        scratch_shapes=[
            pltpu.VMEM((2, SQ, D), jnp.float32),
            pltpu.VMEM((2, SKV, D), jnp.float32),
            pltpu.VMEM((2, SKV, D), jnp.float32),
            pltpu.VMEM((H, SQ, D), jnp.float32),
            pltpu.VMEM((H, SQ, 1), jnp.float32),
            pltpu.VMEM((H, SQ, 1), jnp.float32),
            pltpu.VMEM((H, SQ, D), jnp.bfloat16),
            pltpu.VMEM((2, H, SQ, 1), jnp.float32),
            pltpu.VMEM((3, H, SQ, D), jnp.bfloat16),
            pltpu.VMEM((3, 2, H, SQ, 1), jnp.float32),
            pltpu.SemaphoreType.DMA((2,)),
            pltpu.SemaphoreType.DMA((2, 2)),
            pltpu.SemaphoreType.DMA((2, 2)),
            pltpu.SemaphoreType.DMA((3,)),
            pltpu.SemaphoreType.DMA((3,)),
            pltpu.SemaphoreType.DMA((3,)),
            pltpu.SemaphoreType.DMA((3,)),
            pltpu.SemaphoreType.DMA((7,)),
            pltpu.SemaphoreType.DMA((7,)),
        ],
    )
    return pl.pallas_call(
        _body,
        grid_spec=grid_spec,
        out_shape=jax.ShapeDtypeStruct((B, SQ, H, D), jnp.float32),
        compiler_params=pltpu.CompilerParams(collective_id=0),
    )(rid_arr, Q, K, V)
